# baseline (speedup 1.0000x reference)
import jax
import jax.numpy as jnp
from jax.experimental import pallas as pl
from jax.experimental.pallas import tpu as pltpu

_BT = 1024


def _rot(v, n):
    return jnp.concatenate([v[:, n:], v[:, :n]], axis=1)


def _net_kernel(x_ref, w1_ref, b1_ref, w2_ref, b2_ref,
                wf1_ref, bf1_ref, wf2_ref, bf2_ref, out_ref):
    x = x_ref[...].astype(jnp.bfloat16)
    w1 = w1_ref[...]
    w2 = w2_ref[...]
    b1 = b1_ref[...]
    b2 = b2_ref[...]

    z1_rows = []
    for i in range(6):
        m = jnp.dot(x[:, 112 * i:112 * i + 224], w1,
                    preferred_element_type=jnp.float32)
        for p in range(2):
            mm = jnp.maximum(m[:, 512 * p:512 * p + 256],
                             m[:, 512 * p + 256:512 * p + 512])
            pw = jnp.maximum(mm, _rot(mm, 10))
            z1_rows.append((pw + b1).astype(jnp.bfloat16))
    z1 = jnp.concatenate(z1_rows, axis=1)

    f_rows = []
    for i in range(4):
        ta = jnp.dot(z1[:, 512 * i:512 * i + 1280], w2,
                     preferred_element_type=jnp.float32)
        tb = jnp.dot(z1[:, 512 * i + 256:512 * i + 1536], w2,
                     preferred_element_type=jnp.float32)
        mm = jnp.maximum(ta, tb)
        pw = jnp.maximum(mm, _rot(mm, 20))
        f_rows.append((pw + b2).astype(jnp.bfloat16))
    f = jnp.concatenate(f_rows, axis=1)

    h = jnp.dot(f, wf1_ref[...], preferred_element_type=jnp.float32) + bf1_ref[...]
    y = jnp.dot(h.astype(jnp.bfloat16), wf2_ref[...],
                preferred_element_type=jnp.float32) + bf2_ref[...]
    out_ref[...] = y


def _prep_weights(wb1, b1, wb2, b2, wf1):
    w1big = wb1.reshape(140, 240)
    w1 = jnp.concatenate(
        [jnp.pad(w1big, ((28 * p, 84 - 28 * p), (0, 16))) for p in range(4)],
        axis=1)

    b1u = jnp.pad(b1.reshape(12, 10), ((0, 0), (0, 10))).reshape(1, 240)
    b1u = jnp.pad(b1u, ((0, 0), (0, 16)))
    b2u = jnp.pad(b2.reshape(4, 20), ((0, 0), (0, 20))).reshape(1, 160)
    b2u = jnp.pad(b2u, ((0, 0), (0, 96)))

    w2 = jnp.pad(wb2.reshape(5, 12, 10, 160),
                 ((0, 0), (0, 0), (0, 10), (0, 0))).reshape(5, 240, 160)
    w2 = jnp.pad(w2, ((0, 0), (0, 16), (0, 96))).reshape(1280, 256)

    wf1u = jnp.pad(wf1.reshape(4, 4, 20, 64),
                   ((0, 0), (0, 0), (0, 20), (0, 0))).reshape(4, 160, 64)
    wf1u = jnp.pad(wf1u, ((0, 0), (0, 96), (0, 0))).reshape(1024, 64)
    return w1, b1u, w2, b2u, wf1u


def kernel(wb1, b1, wb2, b2, wf1, bf1, wf2, bf2, x):
    batch = x.shape[0]
    out_sz = 10

    w1, b1u, w2, b2u, wf1u = _prep_weights(wb1, b1, wb2, b2, wf1)
    xb = x.reshape(batch, 784)

    g = -(-batch // _BT)
    bp = g * _BT
    if bp != batch:
        xb = jnp.pad(xb, ((0, bp - batch), (0, 0)))

    out = pl.pallas_call(
        _net_kernel,
        out_shape=jax.ShapeDtypeStruct((bp, 128), jnp.float32),
        grid=(g,),
        in_specs=[
            pl.BlockSpec((_BT, 784), lambda i: (i, 0)),
            pl.BlockSpec(w1.shape, lambda i: (0, 0)),
            pl.BlockSpec(b1u.shape, lambda i: (0, 0)),
            pl.BlockSpec(w2.shape, lambda i: (0, 0)),
            pl.BlockSpec(b2u.shape, lambda i: (0, 0)),
            pl.BlockSpec(wf1u.shape, lambda i: (0, 0)),
            pl.BlockSpec(bf1.shape, lambda i: (0, 0)),
            pl.BlockSpec(wf2.shape, lambda i: (0, 0)),
            pl.BlockSpec(bf2.shape, lambda i: (0, 0)),
        ],
        out_specs=pl.BlockSpec((_BT, 128), lambda i: (i, 0)),
        compiler_params=pltpu.CompilerParams(
            dimension_semantics=("parallel",)),
    )(xb, w1, b1u, w2, b2u, wf1u, bf1, wf2, bf2)

    return out[:batch, :out_sz]


# speedup vs baseline: 1.0006x; 1.0006x over previous
import jax
import jax.numpy as jnp
from jax.experimental import pallas as pl
from jax.experimental.pallas import tpu as pltpu

_BT = 1024


def _rot(v, n):
    return jnp.concatenate([v[:, n:], v[:, :n]], axis=1)


def _net_kernel(x_ref, w1_ref, b1_ref, w2_ref, b2_ref,
                wf1_ref, bf1_ref, wf2_ref, bf2_ref, out_ref):
    x = x_ref[...].astype(jnp.bfloat16)
    w1 = w1_ref[...]
    w2 = w2_ref[...]
    b1 = b1_ref[...]
    b2 = b2_ref[...]

    z1_rows = []
    for i in range(6):
        m = jnp.dot(x[:, 112 * i:112 * i + 224], w1,
                    preferred_element_type=jnp.float32)
        for p in range(2):
            mm = jnp.maximum(m[:, 512 * p:512 * p + 256],
                             m[:, 512 * p + 256:512 * p + 512])
            pw = jnp.maximum(mm, _rot(mm, 10))
            z1_rows.append((pw + b1).astype(jnp.bfloat16))
    z1 = jnp.concatenate(z1_rows, axis=1)

    f_rows = []
    for i in range(4):
        ta = jnp.dot(z1[:, 512 * i:512 * i + 1280], w2,
                     preferred_element_type=jnp.float32)
        tb = jnp.dot(z1[:, 512 * i + 256:512 * i + 1536], w2,
                     preferred_element_type=jnp.float32)
        mm = jnp.maximum(ta, tb)
        pw = jnp.maximum(mm, _rot(mm, 20))
        f_rows.append((pw + b2).astype(jnp.bfloat16))
    f = jnp.concatenate(f_rows, axis=1)

    h = jnp.dot(f, wf1_ref[...], preferred_element_type=jnp.float32) + bf1_ref[...]
    y = jnp.dot(h.astype(jnp.bfloat16), wf2_ref[...],
                preferred_element_type=jnp.float32) + bf2_ref[...]
    out_ref[...] = y[:, :out_ref.shape[1]]


def _prep_weights(wb1, b1, wb2, b2, wf1):
    w1big = wb1.reshape(140, 240)
    w1 = jnp.concatenate(
        [jnp.pad(w1big, ((28 * p, 84 - 28 * p), (0, 16))) for p in range(4)],
        axis=1)

    b1u = jnp.pad(b1.reshape(12, 10), ((0, 0), (0, 10))).reshape(1, 240)
    b1u = jnp.pad(b1u, ((0, 0), (0, 16)))
    b2u = jnp.pad(b2.reshape(4, 20), ((0, 0), (0, 20))).reshape(1, 160)
    b2u = jnp.pad(b2u, ((0, 0), (0, 96)))

    w2 = jnp.pad(wb2.reshape(5, 12, 10, 160),
                 ((0, 0), (0, 0), (0, 10), (0, 0))).reshape(5, 240, 160)
    w2 = jnp.pad(w2, ((0, 0), (0, 16), (0, 96))).reshape(1280, 256)

    wf1u = jnp.pad(wf1.reshape(4, 4, 20, 64),
                   ((0, 0), (0, 0), (0, 20), (0, 0))).reshape(4, 160, 64)
    wf1u = jnp.pad(wf1u, ((0, 0), (0, 96), (0, 0))).reshape(1024, 64)
    return w1, b1u, w2, b2u, wf1u


def kernel(wb1, b1, wb2, b2, wf1, bf1, wf2, bf2, x):
    batch = x.shape[0]
    out_sz = 10

    w1, b1u, w2, b2u, wf1u = _prep_weights(wb1, b1, wb2, b2, wf1)
    xb = x.reshape(batch, 784)

    g = -(-batch // _BT)
    bp = g * _BT
    if bp != batch:
        xb = jnp.pad(xb, ((0, bp - batch), (0, 0)))

    out = pl.pallas_call(
        _net_kernel,
        out_shape=jax.ShapeDtypeStruct((bp, out_sz), jnp.float32),
        grid=(g,),
        in_specs=[
            pl.BlockSpec((_BT, 784), lambda i: (i, 0)),
            pl.BlockSpec(w1.shape, lambda i: (0, 0)),
            pl.BlockSpec(b1u.shape, lambda i: (0, 0)),
            pl.BlockSpec(w2.shape, lambda i: (0, 0)),
            pl.BlockSpec(b2u.shape, lambda i: (0, 0)),
            pl.BlockSpec(wf1u.shape, lambda i: (0, 0)),
            pl.BlockSpec(bf1.shape, lambda i: (0, 0)),
            pl.BlockSpec(wf2.shape, lambda i: (0, 0)),
            pl.BlockSpec(bf2.shape, lambda i: (0, 0)),
        ],
        out_specs=pl.BlockSpec((_BT, out_sz), lambda i: (i, 0)),
        compiler_params=pltpu.CompilerParams(
            dimension_semantics=("parallel",)),
    )(xb, w1, b1u, w2, b2u, wf1u, bf1, wf2, bf2)

    return out[:batch] if bp != batch else out


# speedup vs baseline: 1.4719x; 1.4710x over previous
import jax
import jax.numpy as jnp
from jax.experimental import pallas as pl
from jax.experimental.pallas import tpu as pltpu

_BT = 512


def _rot(v, n):
    return jnp.concatenate([v[:, n:], v[:, :n]], axis=1)


def _net_kernel(x_ref, w1_ref, b1_ref, w2_ref, b2_ref,
                wf1_ref, bf1_ref, wf2_ref, bf2_ref, out_ref):
    bt = x_ref.shape[0]
    x = x_ref[...].astype(jnp.bfloat16).reshape(bt, 784)
    w1 = w1_ref[...]
    w2 = w2_ref[...]
    b1 = b1_ref[...]
    b2 = b2_ref[...]

    z1_rows = []
    for i in range(6):
        m = jnp.dot(x[:, 112 * i:112 * i + 224], w1,
                    preferred_element_type=jnp.float32)
        for p in range(2):
            mm = jnp.maximum(m[:, 512 * p:512 * p + 256],
                             m[:, 512 * p + 256:512 * p + 512])
            pw = jnp.maximum(mm, _rot(mm, 10))
            z1_rows.append((pw + b1).astype(jnp.bfloat16))
    z1 = jnp.concatenate(z1_rows, axis=1)

    f_rows = []
    for i in range(4):
        ta = jnp.dot(z1[:, 512 * i:512 * i + 1280], w2,
                     preferred_element_type=jnp.float32)
        tb = jnp.dot(z1[:, 512 * i + 256:512 * i + 1536], w2,
                     preferred_element_type=jnp.float32)
        mm = jnp.maximum(ta, tb)
        pw = jnp.maximum(mm, _rot(mm, 20))
        f_rows.append((pw + b2).astype(jnp.bfloat16))
    f = jnp.concatenate(f_rows, axis=1)

    h = jnp.dot(f, wf1_ref[...], preferred_element_type=jnp.float32) + bf1_ref[...]
    y = jnp.dot(h.astype(jnp.bfloat16), wf2_ref[...],
                preferred_element_type=jnp.float32) + bf2_ref[...]
    out_ref[...] = y[:, :out_ref.shape[1]]


def _prep_weights(wb1, b1, wb2, b2, wf1):
    w1big = wb1.reshape(140, 240)
    w1 = jnp.concatenate(
        [jnp.pad(w1big, ((28 * p, 84 - 28 * p), (0, 16))) for p in range(4)],
        axis=1)

    b1u = jnp.pad(b1.reshape(12, 10), ((0, 0), (0, 10))).reshape(1, 240)
    b1u = jnp.pad(b1u, ((0, 0), (0, 16)))
    b2u = jnp.pad(b2.reshape(4, 20), ((0, 0), (0, 20))).reshape(1, 160)
    b2u = jnp.pad(b2u, ((0, 0), (0, 96)))

    w2 = jnp.pad(wb2.reshape(5, 12, 10, 160),
                 ((0, 0), (0, 0), (0, 10), (0, 0))).reshape(5, 240, 160)
    w2 = jnp.pad(w2, ((0, 0), (0, 16), (0, 96))).reshape(1280, 256)

    wf1u = jnp.pad(wf1.reshape(4, 4, 20, 64),
                   ((0, 0), (0, 0), (0, 20), (0, 0))).reshape(4, 160, 64)
    wf1u = jnp.pad(wf1u, ((0, 0), (0, 96), (0, 0))).reshape(1024, 64)
    return w1, b1u, w2, b2u, wf1u


def kernel(wb1, b1, wb2, b2, wf1, bf1, wf2, bf2, x):
    batch = x.shape[0]
    out_sz = 10

    w1, b1u, w2, b2u, wf1u = _prep_weights(wb1, b1, wb2, b2, wf1)
    xb = x.reshape(batch, 28, 28)

    g = -(-batch // _BT)
    bp = g * _BT
    if bp != batch:
        xb = jnp.pad(xb, ((0, bp - batch), (0, 0), (0, 0)))

    out = pl.pallas_call(
        _net_kernel,
        out_shape=jax.ShapeDtypeStruct((bp, out_sz), jnp.float32),
        grid=(g,),
        in_specs=[
            pl.BlockSpec((_BT, 28, 28), lambda i: (i, 0, 0)),
            pl.BlockSpec(w1.shape, lambda i: (0, 0)),
            pl.BlockSpec(b1u.shape, lambda i: (0, 0)),
            pl.BlockSpec(w2.shape, lambda i: (0, 0)),
            pl.BlockSpec(b2u.shape, lambda i: (0, 0)),
            pl.BlockSpec(wf1u.shape, lambda i: (0, 0)),
            pl.BlockSpec(bf1.shape, lambda i: (0, 0)),
            pl.BlockSpec(wf2.shape, lambda i: (0, 0)),
            pl.BlockSpec(bf2.shape, lambda i: (0, 0)),
        ],
        out_specs=pl.BlockSpec((_BT, out_sz), lambda i: (i, 0)),
        compiler_params=pltpu.CompilerParams(
            dimension_semantics=("parallel",)),
    )(xb, w1, b1u, w2, b2u, wf1u, bf1, wf2, bf2)

    return out[:batch] if bp != batch else out
